# baseline (speedup 1.0000x reference)
import functools

import jax
import jax.numpy as jnp
from jax import lax
from jax.experimental import pallas as pl
from jax.experimental.pallas import tpu as pltpu
from jax.experimental.pallas import tpu_sc as plsc

_NC = 2
_NS = 16
_L = 16
_NW = _NC * _NS


@functools.lru_cache(maxsize=None)
def _build_quad_tc(H, W, RB):
    nb = H // RB

    def quad_body(g_ref, o_ref):
        i = pl.program_id(0)
        r0 = i * RB
        top = g_ref[pl.ds(r0, RB), :]
        nstart = jnp.minimum(r0 + RB, H - 8)
        nblk = g_ref[pl.ds(nstart, 8), :]
        next_row = jnp.where(i == nb - 1, nblk[7:8, :], nblk[0:1, :])
        bot = jnp.concatenate([top[1:, :], next_row], axis=0)
        tr = jnp.concatenate([top[:, 1:], top[:, -1:]], axis=1)
        br = jnp.concatenate([bot[:, 1:], bot[:, -1:]], axis=1)
        o_ref[...] = jnp.stack([top, tr, bot, br], axis=-1).reshape(RB, 4 * W)

    return pl.pallas_call(
        quad_body,
        grid=(nb,),
        in_specs=[pl.BlockSpec((H, W), lambda i: (0, 0))],
        out_specs=pl.BlockSpec((RB, 4 * W), lambda i: (i, 0)),
        out_shape=jax.ShapeDtypeStruct((H, 4 * W), jnp.float32),
    )


@functools.lru_cache(maxsize=None)
def _build_sc(P, H, W, C):
    PW = P // _NW
    n_chunks = PW // C
    mesh = plsc.VectorSubcoreMesh(
        core_axis_name="c", subcore_axis_name="s",
        num_cores=_NC, num_subcores=_NS)

    vmem_f = lambda n: pltpu.VMEM((n,), jnp.float32)
    vmem_i = lambda n: pltpu.VMEM((n,), jnp.int32)

    @functools.partial(
        pl.kernel,
        out_type=jax.ShapeDtypeStruct((P,), jnp.float32),
        mesh=mesh,
        scratch_types=[
            [vmem_f(C)] * 2,
            [vmem_f(C)] * 2,
            [vmem_i(4 * C)] * 2,
            [vmem_f(4 * C)] * 2,
            [vmem_f(C)] * 2,
            [vmem_f(C)] * 2,
            [vmem_f(C)] * 2,
            [pltpu.SemaphoreType.DMA] * 2,
            [pltpu.SemaphoreType.DMA] * 2,
        ],
    )
    def grid_sample(xy_hbm, t4_hbm, out_hbm, xv, yv, idxv, valv, wxv, wyv,
                    outv, sx, sg):
        wid = lax.axis_index("s") * _NC + lax.axis_index("c")
        base0 = wid * PW
        fw = jnp.float32(W - 1)
        fh = jnp.float32(H - 1)
        lane = lax.iota(jnp.int32, _L)

        def _vperm(v, idx):
            dn = lax.GatherDimensionNumbers(
                offset_dims=(), collapsed_slice_dims=(0,),
                start_index_map=(0,))
            return lax.gather(v, idx[:, None], dn, slice_sizes=(1,),
                              mode=lax.GatherScatterMode.PROMISE_IN_BOUNDS)

        def start_load(k, b):
            base = base0 + k * C
            a = pltpu.async_copy(xy_hbm.at[0, pl.ds(base, C)], xv[b], sx[b])
            c = pltpu.async_copy(xy_hbm.at[1, pl.ds(base, C)], yv[b], sx[b])
            return (a, c)

        def compute_idx(b):
            @pl.loop(0, C // _L)
            def _indices(j):
                o = j * _L
                xf = xv[b][pl.ds(o, _L)] * fw
                yf = yv[b][pl.ds(o, _L)] * fh
                x0 = jnp.clip(xf.astype(jnp.int32), 0, W - 1)
                y0 = jnp.clip(yf.astype(jnp.int32), 0, H - 1)
                wxv[b][pl.ds(o, _L)] = xf - x0.astype(jnp.float32)
                wyv[b][pl.ds(o, _L)] = yf - y0.astype(jnp.float32)
                q4 = (y0 * W + x0) * 4
                psel = lane >> 2
                csel = lane & 3
                for q in range(4):
                    quad = _vperm(q4, psel + 4 * q) + csel
                    idxv[b][pl.ds(4 * o + q * _L, _L)] = quad

        def start_gather(b):
            return pltpu.async_copy(t4_hbm.at[idxv[b]], valv[b], sg[b])

        def blend(b):
            @pl.loop(0, C // _L)
            def _blend(j):
                o = j * _L
                quads = [valv[b][pl.ds(4 * o + q * _L, _L)] for q in range(4)]
                csel = lane & 3
                v = []
                for c in range(4):
                    g = [_vperm(qv, csel * 4 + c) for qv in quads]
                    v.append(jnp.where(
                        lane < 4, g[0],
                        jnp.where(lane < 8, g[1],
                                  jnp.where(lane < 12, g[2], g[3]))))
                wx = wxv[b][pl.ds(o, _L)]
                wy = wyv[b][pl.ds(o, _L)]
                top = v[0] + wx * (v[1] - v[0])
                bot = v[2] + wx * (v[3] - v[2])
                outv[b][pl.ds(o, _L)] = top + wy * (bot - top)

        def store(k, b):
            base = base0 + k * C
            pltpu.sync_copy(outv[b], out_hbm.at[pl.ds(base, C)])

        loads = [None] * n_chunks
        gathers = [None] * n_chunks
        loads[0] = start_load(0, 0)
        for k in range(n_chunks):
            b = k % 2
            for d in loads[k]:
                d.wait()
            if k + 1 < n_chunks:
                loads[k + 1] = start_load(k + 1, 1 - b)
            compute_idx(b)
            if k >= 1:
                gathers[k - 1].wait()
            gathers[k] = start_gather(b)
            if k >= 1:
                blend(1 - b)
                store(k - 1, 1 - b)
        gathers[n_chunks - 1].wait()
        blend((n_chunks - 1) % 2)
        store(n_chunks - 1, (n_chunks - 1) % 2)

    return grid_sample


def kernel(xy, grid):
    P = xy.shape[0]
    H, W = grid.shape[-2], grid.shape[-1]
    g = grid.reshape(H, W)
    gx = jnp.concatenate([g[:, 1:], g[:, -1:]], axis=1)
    gy = jnp.concatenate([g[1:, :], g[-1:, :]], axis=0)
    gxy = jnp.concatenate([gy[:, 1:], gy[:, -1:]], axis=1)
    t4 = jnp.stack([g, gx, gy, gxy], axis=-1)
    return _build_sc(P, H, W, 2048)(xy.T, t4.reshape(-1))


# speedup vs baseline: 3.8209x; 3.8209x over previous
import functools

import jax
import jax.numpy as jnp
from jax import lax
from jax.experimental import pallas as pl
from jax.experimental.pallas import tpu as pltpu
from jax.experimental.pallas import tpu_sc as plsc

_NC = 2
_NS = 16
_L = 16
_NW = _NC * _NS


@functools.lru_cache(maxsize=None)
def _build(P, H, W, C):
    PW = P // _NW
    n_chunks = PW // C
    mesh = plsc.VectorSubcoreMesh(
        core_axis_name="c", subcore_axis_name="s",
        num_cores=_NC, num_subcores=_NS)

    vmem_f = lambda n: pltpu.VMEM((n,), jnp.float32)
    vmem_i = lambda n: pltpu.VMEM((n,), jnp.int32)

    @functools.partial(
        pl.kernel,
        out_type=jax.ShapeDtypeStruct((P,), jnp.float32),
        mesh=mesh,
        scratch_types=[
            [vmem_f(2 * C)] * 2,
            [vmem_i(4 * C)] * 2,
            [vmem_f(4 * C)] * 2,
            [vmem_f(C)] * 2,
            [vmem_f(C)] * 2,
            [vmem_f(C)] * 2,
            [pltpu.SemaphoreType.DMA] * 2,
            [pltpu.SemaphoreType.DMA] * 2,
            [pltpu.SemaphoreType.DMA] * 2,
        ],
    )
    def grid_sample(xy_hbm, g_hbm, out_hbm, xyv, idxv, valv, wxv, wyv,
                    outv, sx, sg, so):
        wid = lax.axis_index("s") * _NC + lax.axis_index("c")
        base0 = wid * PW
        fw = jnp.float32(W - 1)
        fh = jnp.float32(H - 1)
        lane = lax.iota(jnp.int32, _L)

        def _vperm(v, idx):
            dn = lax.GatherDimensionNumbers(
                offset_dims=(), collapsed_slice_dims=(0,),
                start_index_map=(0,))
            return lax.gather(v, idx[:, None], dn, slice_sizes=(1,),
                              mode=lax.GatherScatterMode.PROMISE_IN_BOUNDS)

        def start_load(k, b):
            base = base0 + k * C
            return pltpu.async_copy(
                xy_hbm.at[pl.ds(2 * base, 2 * C)], xyv[b], sx[b])

        def compute_idx(b):
            @pl.loop(0, C // _L)
            def _indices(j):
                o = j * _L
                av = xyv[b][pl.ds(2 * o, _L)]
                bv = xyv[b][pl.ds(2 * o + _L, _L)]
                ie = (lane + lane) & 15
                io = ie + 1
                xs = jnp.where(lane < 8, _vperm(av, ie), _vperm(bv, ie))
                ys = jnp.where(lane < 8, _vperm(av, io), _vperm(bv, io))
                xf = xs * fw
                yf = ys * fh
                x0 = jnp.clip(xf.astype(jnp.int32), 0, W - 1)
                y0 = jnp.clip(yf.astype(jnp.int32), 0, H - 1)
                wxv[b][pl.ds(o, _L)] = xf - x0.astype(jnp.float32)
                wyv[b][pl.ds(o, _L)] = yf - y0.astype(jnp.float32)
                x1 = jnp.minimum(x0 + 1, W - 1)
                r0 = y0 * W
                r1 = jnp.minimum(y0 + 1, H - 1) * W
                corner = [r0 + x0, r0 + x1, r1 + x0, r1 + x1]
                psel = lane >> 2
                csel = lane & 3
                for q in range(4):
                    g = [_vperm(cv, psel + 4 * q) for cv in corner]
                    quad = jnp.where(
                        csel == 0, g[0],
                        jnp.where(csel == 1, g[1],
                                  jnp.where(csel == 2, g[2], g[3])))
                    idxv[b][pl.ds(4 * o + q * _L, _L)] = quad

        def start_gather(b):
            return pltpu.async_copy(g_hbm.at[idxv[b]], valv[b], sg[b])

        def blend(b):
            @pl.loop(0, C // _L)
            def _blend(j):
                o = j * _L
                quads = [valv[b][pl.ds(4 * o + q * _L, _L)] for q in range(4)]
                csel = lane & 3
                v = []
                for c in range(4):
                    g = [_vperm(qv, csel * 4 + c) for qv in quads]
                    v.append(jnp.where(
                        lane < 4, g[0],
                        jnp.where(lane < 8, g[1],
                                  jnp.where(lane < 12, g[2], g[3]))))
                wx = wxv[b][pl.ds(o, _L)]
                wy = wyv[b][pl.ds(o, _L)]
                top = v[0] + wx * (v[1] - v[0])
                bot = v[2] + wx * (v[3] - v[2])
                outv[b][pl.ds(o, _L)] = top + wy * (bot - top)

        def start_store(k, b):
            base = base0 + k * C
            return pltpu.async_copy(outv[b], out_hbm.at[pl.ds(base, C)],
                                    so[b])

        loads = [None] * n_chunks
        gathers = [None] * n_chunks
        store_desc = [None, None]
        loads[0] = start_load(0, 0)
        for k in range(n_chunks):
            b = k % 2
            loads[k].wait()
            if k + 1 < n_chunks:
                loads[k + 1] = start_load(k + 1, 1 - b)
            compute_idx(b)
            if k >= 1:
                gathers[k - 1].wait()
            gathers[k] = start_gather(b)
            if k >= 1:
                if store_desc[1 - b] is not None:
                    store_desc[1 - b].wait()
                blend(1 - b)
                store_desc[1 - b] = start_store(k - 1, 1 - b)
        bl = (n_chunks - 1) % 2
        gathers[n_chunks - 1].wait()
        if store_desc[bl] is not None:
            store_desc[bl].wait()
        blend(bl)
        start_store(n_chunks - 1, bl).wait()
        if store_desc[1 - bl] is not None:
            store_desc[1 - bl].wait()

    return grid_sample


def kernel(xy, grid):
    P = xy.shape[0]
    H, W = grid.shape[-2], grid.shape[-1]
    return _build(P, H, W, 2048)(xy.reshape(-1), grid.reshape(-1))


# speedup vs baseline: 29.0943x; 7.6146x over previous
import functools

import jax
import jax.numpy as jnp
from jax import lax
from jax.experimental import pallas as pl
from jax.experimental.pallas import tpu as pltpu
from jax.experimental.pallas import tpu_sc as plsc

_NC = 2
_NS = 16
_L = 16
_NW = _NC * _NS


@functools.lru_cache(maxsize=None)
def _build(P, H, W, C):
    PW = P // _NW
    n_chunks = PW // C
    mesh = plsc.VectorSubcoreMesh(
        core_axis_name="c", subcore_axis_name="s",
        num_cores=_NC, num_subcores=_NS)

    vmem_f = lambda n: pltpu.VMEM((n,), jnp.float32)
    vmem_i = lambda n: pltpu.VMEM((n,), jnp.int32)

    @functools.partial(
        pl.kernel,
        out_type=jax.ShapeDtypeStruct((P,), jnp.float32),
        mesh=mesh,
        scratch_types=[
            [[vmem_f(C), vmem_f(C)]] * 2,
            [vmem_i(4 * C)] * 2,
            [vmem_f(4 * C)] * 2,
            [vmem_f(C)] * 2,
            [vmem_f(C)] * 2,
            [vmem_f(C)] * 2,
            [pltpu.SemaphoreType.DMA] * 2,
            [pltpu.SemaphoreType.DMA] * 2,
            [pltpu.SemaphoreType.DMA] * 2,
        ],
    )
    def grid_sample(xy_hbm, g_hbm, out_hbm, xyv, idxv, valv, wxv, wyv,
                    outv, sx, sg, so):
        wid = lax.axis_index("s") * _NC + lax.axis_index("c")
        base0 = wid * PW
        fw = jnp.float32(W - 1)
        fh = jnp.float32(H - 1)
        lane = lax.iota(jnp.int32, _L)

        def _vperm(v, idx):
            dn = lax.GatherDimensionNumbers(
                offset_dims=(), collapsed_slice_dims=(0,),
                start_index_map=(0,))
            return lax.gather(v, idx[:, None], dn, slice_sizes=(1,),
                              mode=lax.GatherScatterMode.PROMISE_IN_BOUNDS)

        def start_load(k, b):
            base = base0 + k * C
            a = pltpu.async_copy(xy_hbm.at[0, pl.ds(base, C)], xyv[b][0],
                                 sx[b])
            c = pltpu.async_copy(xy_hbm.at[1, pl.ds(base, C)], xyv[b][1],
                                 sx[b])
            return (a, c)

        def compute_idx(b):
            @pl.loop(0, C // _L)
            def _indices(j):
                o = j * _L
                xf = xyv[b][0][pl.ds(o, _L)] * fw
                yf = xyv[b][1][pl.ds(o, _L)] * fh
                x0 = jnp.clip(xf.astype(jnp.int32), 0, W - 1)
                y0 = jnp.clip(yf.astype(jnp.int32), 0, H - 1)
                wxv[b][pl.ds(o, _L)] = xf - x0.astype(jnp.float32)
                wyv[b][pl.ds(o, _L)] = yf - y0.astype(jnp.float32)
                x1 = jnp.minimum(x0 + 1, W - 1)
                r0 = y0 * W
                r1 = jnp.minimum(y0 + 1, H - 1) * W
                corner = [r0 + x0, r0 + x1, r1 + x0, r1 + x1]
                psel = lane >> 2
                csel = lane & 3
                for q in range(4):
                    g = [_vperm(cv, psel + 4 * q) for cv in corner]
                    quad = jnp.where(
                        csel == 0, g[0],
                        jnp.where(csel == 1, g[1],
                                  jnp.where(csel == 2, g[2], g[3])))
                    idxv[b][pl.ds(4 * o + q * _L, _L)] = quad

        def start_gather(b):
            return pltpu.async_copy(g_hbm.at[idxv[b]], valv[b], sg[b])

        def blend(b):
            @pl.loop(0, C // _L)
            def _blend(j):
                o = j * _L
                quads = [valv[b][pl.ds(4 * o + q * _L, _L)] for q in range(4)]
                csel = lane & 3
                v = []
                for c in range(4):
                    g = [_vperm(qv, csel * 4 + c) for qv in quads]
                    v.append(jnp.where(
                        lane < 4, g[0],
                        jnp.where(lane < 8, g[1],
                                  jnp.where(lane < 12, g[2], g[3]))))
                wx = wxv[b][pl.ds(o, _L)]
                wy = wyv[b][pl.ds(o, _L)]
                top = v[0] + wx * (v[1] - v[0])
                bot = v[2] + wx * (v[3] - v[2])
                outv[b][pl.ds(o, _L)] = top + wy * (bot - top)

        def start_store(k, b):
            base = base0 + k * C
            return pltpu.async_copy(outv[b], out_hbm.at[pl.ds(base, C)],
                                    so[b])

        loads = [None] * n_chunks
        gathers = [None] * n_chunks
        store_desc = [None, None]
        loads[0] = start_load(0, 0)
        for k in range(n_chunks):
            b = k % 2
            for d in loads[k]:
                d.wait()
            if k + 1 < n_chunks:
                loads[k + 1] = start_load(k + 1, 1 - b)
            compute_idx(b)
            if k >= 1:
                gathers[k - 1].wait()
            gathers[k] = start_gather(b)
            if k >= 1:
                if store_desc[1 - b] is not None:
                    store_desc[1 - b].wait()
                blend(1 - b)
                store_desc[1 - b] = start_store(k - 1, 1 - b)
        bl = (n_chunks - 1) % 2
        gathers[n_chunks - 1].wait()
        if store_desc[bl] is not None:
            store_desc[bl].wait()
        blend(bl)
        start_store(n_chunks - 1, bl).wait()
        if store_desc[1 - bl] is not None:
            store_desc[1 - bl].wait()

    return grid_sample


def kernel(xy, grid):
    P = xy.shape[0]
    H, W = grid.shape[-2], grid.shape[-1]
    return _build(P, H, W, 2048)(xy.T, grid.reshape(-1))


# speedup vs baseline: 29.3964x; 1.0104x over previous
import functools

import jax
import jax.numpy as jnp
from jax import lax
from jax.experimental import pallas as pl
from jax.experimental.pallas import tpu as pltpu
from jax.experimental.pallas import tpu_sc as plsc

_NC = 2
_NS = 16
_L = 16
_NW = _NC * _NS


@functools.lru_cache(maxsize=None)
def _build(P, H, W, C):
    PW = P // _NW
    n_chunks = PW // C
    mesh = plsc.VectorSubcoreMesh(
        core_axis_name="c", subcore_axis_name="s",
        num_cores=_NC, num_subcores=_NS)

    vmem_f = lambda n: pltpu.VMEM((n,), jnp.float32)
    vmem_i = lambda n: pltpu.VMEM((n,), jnp.int32)

    @functools.partial(
        pl.kernel,
        out_type=jax.ShapeDtypeStruct((P,), jnp.float32),
        mesh=mesh,
        scratch_types=[
            [[vmem_f(C), vmem_f(C)]] * 2,
            [vmem_i(4 * C)] * 2,
            [vmem_f(4 * C)] * 2,
            [vmem_f(C)] * 2,
            [vmem_f(C)] * 2,
            [vmem_f(C)] * 2,
            [pltpu.SemaphoreType.DMA] * 2,
            [pltpu.SemaphoreType.DMA] * 2,
            [pltpu.SemaphoreType.DMA] * 2,
        ],
    )
    def grid_sample(xy_hbm, g_hbm, out_hbm, xyv, idxv, valv, wxv, wyv,
                    outv, sx, sg, so):
        wid = lax.axis_index("s") * _NC + lax.axis_index("c")
        base0 = wid * PW
        fw = jnp.float32(W - 1)
        fh = jnp.float32(H - 1)
        lane = lax.iota(jnp.int32, _L)

        def _vperm(v, idx):
            dn = lax.GatherDimensionNumbers(
                offset_dims=(), collapsed_slice_dims=(0,),
                start_index_map=(0,))
            return lax.gather(v, idx[:, None], dn, slice_sizes=(1,),
                              mode=lax.GatherScatterMode.PROMISE_IN_BOUNDS)

        def start_load(k, b):
            base = base0 + k * C
            a = pltpu.async_copy(xy_hbm.at[0, pl.ds(base, C)], xyv[b][0],
                                 sx[b])
            c = pltpu.async_copy(xy_hbm.at[1, pl.ds(base, C)], xyv[b][1],
                                 sx[b])
            return (a, c)

        def compute_idx(b):
            @pl.loop(0, C // _L)
            def _indices(j):
                o = j * _L
                xf = xyv[b][0][pl.ds(o, _L)] * fw
                yf = xyv[b][1][pl.ds(o, _L)] * fh
                x0 = jnp.clip(xf.astype(jnp.int32), 0, W - 1)
                y0 = jnp.clip(yf.astype(jnp.int32), 0, H - 1)
                wxv[b][pl.ds(o, _L)] = xf - x0.astype(jnp.float32)
                wyv[b][pl.ds(o, _L)] = yf - y0.astype(jnp.float32)
                x1 = jnp.minimum(x0 + 1, W - 1)
                r0 = y0 * W
                r1 = jnp.minimum(y0 + 1, H - 1) * W
                corner = [r0 + x0, r0 + x1, r1 + x0, r1 + x1]
                psel = lane >> 2
                csel = lane & 3
                for q in range(4):
                    g = [_vperm(cv, psel + 4 * q) for cv in corner]
                    quad = jnp.where(
                        csel == 0, g[0],
                        jnp.where(csel == 1, g[1],
                                  jnp.where(csel == 2, g[2], g[3])))
                    idxv[b][pl.ds(4 * o + q * _L, _L)] = quad

        def start_gather(b):
            return pltpu.async_copy(g_hbm.at[idxv[b]], valv[b], sg[b])

        def blend(b):
            @pl.loop(0, C // _L)
            def _blend(j):
                o = j * _L
                quads = [valv[b][pl.ds(4 * o + q * _L, _L)] for q in range(4)]
                csel = lane & 3
                v = []
                for c in range(4):
                    g = [_vperm(qv, csel * 4 + c) for qv in quads]
                    v.append(jnp.where(
                        lane < 4, g[0],
                        jnp.where(lane < 8, g[1],
                                  jnp.where(lane < 12, g[2], g[3]))))
                wx = wxv[b][pl.ds(o, _L)]
                wy = wyv[b][pl.ds(o, _L)]
                top = v[0] + wx * (v[1] - v[0])
                bot = v[2] + wx * (v[3] - v[2])
                outv[b][pl.ds(o, _L)] = top + wy * (bot - top)

        def start_store(k, b):
            base = base0 + k * C
            return pltpu.async_copy(outv[b], out_hbm.at[pl.ds(base, C)],
                                    so[b])

        loads = [None] * n_chunks
        gathers = [None] * n_chunks
        store_desc = [None, None]
        loads[0] = start_load(0, 0)
        for k in range(n_chunks):
            b = k % 2
            for d in loads[k]:
                d.wait()
            if k + 1 < n_chunks:
                loads[k + 1] = start_load(k + 1, 1 - b)
            compute_idx(b)
            if k >= 1:
                gathers[k - 1].wait()
            gathers[k] = start_gather(b)
            if k >= 1:
                if store_desc[1 - b] is not None:
                    store_desc[1 - b].wait()
                blend(1 - b)
                store_desc[1 - b] = start_store(k - 1, 1 - b)
        bl = (n_chunks - 1) % 2
        gathers[n_chunks - 1].wait()
        if store_desc[bl] is not None:
            store_desc[bl].wait()
        blend(bl)
        start_store(n_chunks - 1, bl).wait()
        if store_desc[1 - bl] is not None:
            store_desc[1 - bl].wait()

    return grid_sample


def kernel(xy, grid):
    P = xy.shape[0]
    H, W = grid.shape[-2], grid.shape[-1]
    return _build(P, H, W, 4096)(xy.T, grid.reshape(-1))
